# baseline (speedup 1.0000x reference)
import jax
import jax.numpy as jnp
from jax.experimental import pallas as pl

S = 64
TOPK = 8
THRESH = 0.5
CHUNK = 512

_F32 = jnp.float32


def _dot_t0_hi(a, b):
    return jax.lax.dot_general(a, b, (((0,), (0,)), ((), ())),
                               precision=jax.lax.Precision.HIGHEST,
                               preferred_element_type=_F32)


def _dot_t1(a, b):
    return jax.lax.dot_general(a, b, (((1,), (1,)), ((), ())),
                               preferred_element_type=_F32)


def _dot_t1_hi(a, b):
    return jax.lax.dot_general(a, b, (((1,), (1,)), ((), ())),
                               precision=jax.lax.Precision.HIGHEST,
                               preferred_element_type=_F32)


def _dot_hi(a, b):
    return jax.lax.dot_general(a, b, (((1,), (0,)), ((), ())),
                               precision=jax.lax.Precision.HIGHEST,
                               preferred_element_type=_F32)


def _shift_down(x, n):
    idx = jax.lax.broadcasted_iota(jnp.int32, x.shape, 0)
    r = jnp.roll(x, n, axis=0)
    return jnp.where(idx < n, jnp.zeros_like(x), r)


def _shift_up(x, n):
    L = x.shape[0]
    idx = jax.lax.broadcasted_iota(jnp.int32, x.shape, 0)
    r = jnp.roll(x, -n, axis=0)
    return jnp.where(idx >= L - n, jnp.zeros_like(x), r)


def _cplx_weight(Wr, Wi):
    row0 = jnp.stack([Wr, -Wi], axis=-1)
    row1 = jnp.stack([Wi, Wr], axis=-1)
    A = jnp.stack([row0, row1], axis=1)
    O, _, D, _ = A.shape
    return A.reshape(2 * O, 2 * D)


def _events_body(zc_ref, ws_ref, ak_ref, av_ref, sb_ref, ns_ref,
                 sal_ref, nk_ref, nv_ref, mask_ref):
    zc = zc_ref[0]
    L, D2 = zc.shape

    zc_b = zc.astype(jnp.bfloat16).astype(_F32)
    ws_b = ws_ref[...].astype(jnp.bfloat16).astype(_F32)
    pr = jnp.sum(zc_b * ws_b[0:1, :], axis=1, keepdims=True)
    pi = jnp.sum(zc_b * ws_b[1:2, :], axis=1, keepdims=True)
    phase = jnp.sqrt(pr * pr + pi * pi + 1e-12)

    sq = zc * zc
    pair = sq + jnp.roll(sq, -1, axis=1)
    mag = jnp.sqrt(pair + 1e-12)
    lane = jax.lax.broadcasted_iota(jnp.int32, (L, D2), 1)
    even = (lane % 2) == 0
    avg = jnp.sum(jnp.where(even, mag, 0.0), axis=1,
                  keepdims=True) * (2.0 / D2)
    local = (_shift_down(avg, 1) + _shift_down(avg, 2) + avg
             + _shift_up(avg, 1) + _shift_up(avg, 2)) / 5.0
    novelty = (avg - local) * ns_ref[0, 0]
    sal = jax.nn.sigmoid(phase + novelty + sb_ref[0, 0])

    above = (sal > THRESH).astype(jnp.int32)
    prev = _shift_down(above, 1)
    starts = above * (1 - prev)
    csum = starts
    d = 1
    while d < L:
        csum = csum + _shift_down(csum, d)
        d *= 2
    span = csum - 1
    seg = jnp.where((above > 0) & (span < S), span, S)

    iota_s = jax.lax.broadcasted_iota(jnp.int32, (L, S), 1)
    onehot = (seg == iota_s).astype(_F32)
    zw = zc * sal
    num = _dot_t0_hi(onehot, zw)
    den = _dot_t0_hi(onehot, sal)
    cnt = _dot_t0_hi(onehot, jnp.ones_like(sal))
    ev = num / jnp.maximum(den, 1e-8)
    mcol = (cnt > 0).astype(_F32)

    nk_ref[0] = mcol * _dot_t1(ev, ak_ref[...])
    nv_ref[0] = mcol * _dot_t1(ev, av_ref[...])
    mask_ref[0] = (jnp.sum(onehot, axis=0, keepdims=True) > 0).astype(_F32)
    sal_ref[0] = sal


def _read_body(zc_ref, aq_ref, k_ref, v_ref, mask_ref, g_ref, out_ref):
    zc = zc_ref[0]
    C, D2 = zc.shape
    q = _dot_t1(zc, aq_ref[...])

    kk = k_ref[0]
    dot = _dot_t1(q, kk)
    qmag = jnp.sqrt(jnp.sum(q * q, axis=1, keepdims=True) + 1e-8)
    kmag = jnp.sqrt(_dot_t1_hi(jnp.ones((1, D2), _F32), kk * kk) + 1e-8)
    scores = dot / (qmag * kmag + 1e-8)
    scores = jnp.where(mask_ref[0] == 0.0, -1e9, scores)

    iota_s = jax.lax.broadcasted_iota(jnp.int32, (C, S), 1)
    work = scores
    sel = []
    onehots = []
    for _ in range(TOPK):
        m = jnp.max(work, axis=1, keepdims=True)
        ismax = work == m
        idx = jnp.min(jnp.where(ismax, iota_s, S), axis=1, keepdims=True)
        oh = iota_s == idx
        sel.append(m)
        onehots.append(oh.astype(_F32))
        work = jnp.where(oh, -3.4e38, work)

    sel8 = jnp.concatenate(sel, axis=1)
    mx = jnp.max(sel8, axis=1, keepdims=True)
    e = jnp.exp(sel8 - mx)
    wts = e / jnp.sum(e, axis=1, keepdims=True)
    attn = wts[:, 0:1] * onehots[0]
    for j in range(1, TOPK):
        attn = attn + wts[:, j:j + 1] * onehots[j]

    ret = _dot_hi(attn, v_ref[0])
    rms = jnp.sqrt(jnp.sum(ret * ret, axis=1, keepdims=True) * (2.0 / D2)
                   + 1e-8)
    out_ref[0] = ret / rms * g_ref[...]


@jax.jit
def kernel(z, Ws_r, Ws_i, Wk_r, Wk_i, Wv_r, Wv_i, Wq_r, Wq_i,
           score_bias, novelty_scale, gamma):
    B, L, D, _ = z.shape
    D2 = 2 * D
    zc = z.reshape(B, L, D2)
    ws = _cplx_weight(Ws_r, Ws_i)
    ak = _cplx_weight(Wk_r, Wk_i)
    av = _cplx_weight(Wv_r, Wv_i)
    aq = _cplx_weight(Wq_r, Wq_i)
    sb = jnp.reshape(score_bias, (1, 1)).astype(_F32)
    ns = jnp.reshape(novelty_scale, (1, 1)).astype(_F32)
    gi = jnp.stack([gamma, gamma], axis=-1).reshape(1, D2).astype(_F32)

    full = lambda shape: pl.BlockSpec(shape, lambda b: (0,) * len(shape))
    row3 = lambda shape: pl.BlockSpec(shape, lambda b: (b, 0, 0))

    sal3, nk, nv, mask3 = pl.pallas_call(
        _events_body,
        grid=(B,),
        in_specs=[
            row3((1, L, D2)),
            full((2, D2)),
            full((D2, D2)), full((D2, D2)),
            full((1, 1)), full((1, 1)),
        ],
        out_specs=[
            row3((1, L, 1)),
            row3((1, S, D2)), row3((1, S, D2)),
            row3((1, 1, S)),
        ],
        out_shape=[
            jax.ShapeDtypeStruct((B, L, 1), _F32),
            jax.ShapeDtypeStruct((B, S, D2), _F32),
            jax.ShapeDtypeStruct((B, S, D2), _F32),
            jax.ShapeDtypeStruct((B, 1, S), _F32),
        ],
    )(zc, ws, ak, av, sb, ns)

    nc = L // CHUNK
    chunk3 = lambda shape: pl.BlockSpec(shape, lambda b, c: (b, c, 0))
    bcast3 = lambda shape: pl.BlockSpec(shape, lambda b, c: (b, 0, 0))
    full2 = lambda shape: pl.BlockSpec(shape, lambda b, c: (0,) * len(shape))

    out_i = pl.pallas_call(
        _read_body,
        grid=(B, nc),
        in_specs=[
            chunk3((1, CHUNK, D2)),
            full2((D2, D2)),
            bcast3((1, S, D2)), bcast3((1, S, D2)),
            bcast3((1, 1, S)),
            full2((1, D2)),
        ],
        out_specs=[chunk3((1, CHUNK, D2))],
        out_shape=[jax.ShapeDtypeStruct((B, L, D2), _F32)],
    )(zc, aq, nk, nv, mask3, gi)[0]

    out = out_i.reshape(B, L, D, 2)
    new_keys = nk.reshape(B, S, D, 2)
    new_values = nv.reshape(B, S, D, 2)
    new_mask = mask3[:, 0, :]
    salience = sal3[:, :, 0]
    return out, new_keys, new_values, new_mask, salience


# speedup vs baseline: 6.8440x; 6.8440x over previous
import jax
import jax.numpy as jnp
from jax.experimental import pallas as pl

S = 64
TOPK = 8
THRESH = 0.5
CHUNK = 1024

_F32 = jnp.float32


def _dot_t0_hi(a, b):
    return jax.lax.dot_general(a, b, (((0,), (0,)), ((), ())),
                               precision=jax.lax.Precision.HIGHEST,
                               preferred_element_type=_F32)


def _dot_t1(a, b):
    return jax.lax.dot_general(a, b, (((1,), (1,)), ((), ())),
                               preferred_element_type=_F32)


def _dot_t1_hi(a, b):
    return jax.lax.dot_general(a, b, (((1,), (1,)), ((), ())),
                               precision=jax.lax.Precision.HIGHEST,
                               preferred_element_type=_F32)


def _dot_hi(a, b):
    return jax.lax.dot_general(a, b, (((1,), (0,)), ((), ())),
                               precision=jax.lax.Precision.HIGHEST,
                               preferred_element_type=_F32)


def _shift_down(x, n):
    idx = jax.lax.broadcasted_iota(jnp.int32, x.shape, 0)
    r = jnp.roll(x, n, axis=0)
    return jnp.where(idx < n, jnp.zeros_like(x), r)


def _shift_up(x, n):
    L = x.shape[0]
    idx = jax.lax.broadcasted_iota(jnp.int32, x.shape, 0)
    r = jnp.roll(x, -n, axis=0)
    return jnp.where(idx >= L - n, jnp.zeros_like(x), r)


def _events_body(zr_ref, zi_ref, ws8_ref, wkr_ref, wki_ref,
                 wvr_ref, wvi_ref, sb_ref, ns_ref,
                 sal_ref, nkr_ref, nki_ref, nvr_ref, nvi_ref, mask_ref):
    zr = zr_ref[0]
    zi = zi_ref[0]
    L, D = zr.shape

    ws8 = ws8_ref[...]
    pzr = _dot_t1(zr, ws8)
    pzi = _dot_t1(zi, ws8)
    pr = pzr[:, 0:1] - pzi[:, 1:2]
    pi = pzr[:, 1:2] + pzi[:, 0:1]
    phase = jnp.sqrt(pr * pr + pi * pi + 1e-12)
    mag = jnp.sqrt(zr * zr + zi * zi + 1e-12)
    avg = jnp.mean(mag, axis=1, keepdims=True)
    local = (_shift_down(avg, 1) + _shift_down(avg, 2) + avg
             + _shift_up(avg, 1) + _shift_up(avg, 2)) / 5.0
    novelty = (avg - local) * ns_ref[0, 0]
    sal = jax.nn.sigmoid(phase + novelty + sb_ref[0, 0])

    above = (sal > THRESH).astype(jnp.int32)
    prev = _shift_down(above, 1)
    starts = above * (1 - prev)
    csum = starts
    d = 1
    while d < L:
        csum = csum + _shift_down(csum, d)
        d *= 2
    span = csum - 1
    seg = jnp.where((above > 0) & (span < S), span, S)

    iota_s = jax.lax.broadcasted_iota(jnp.int32, (L, S), 1)
    onehot = (seg == iota_s).astype(_F32)
    zwr = zr * sal
    zwi = zi * sal
    numr = _dot_t0_hi(onehot, zwr)
    numi = _dot_t0_hi(onehot, zwi)
    den = _dot_t0_hi(onehot, sal)
    cnt = _dot_t0_hi(onehot, jnp.ones_like(sal))
    dsafe = jnp.maximum(den, 1e-8)
    evr = numr / dsafe
    evi = numi / dsafe
    mcol = (cnt > 0).astype(_F32)

    wkr = wkr_ref[...]
    wki = wki_ref[...]
    wvr = wvr_ref[...]
    wvi = wvi_ref[...]
    nkr_ref[0] = mcol * (_dot_t1(evr, wkr) - _dot_t1(evi, wki))
    nki_ref[0] = mcol * (_dot_t1(evr, wki) + _dot_t1(evi, wkr))
    nvr_ref[0] = mcol * (_dot_t1(evr, wvr) - _dot_t1(evi, wvi))
    nvi_ref[0] = mcol * (_dot_t1(evr, wvi) + _dot_t1(evi, wvr))
    mask_ref[0] = (jnp.sum(onehot, axis=0, keepdims=True) > 0).astype(_F32)
    sal_ref[0] = sal


def _read_body(zr_ref, zi_ref, wqr_ref, wqi_ref, kr_ref, ki_ref,
               vr_ref, vi_ref, mask_ref, g_ref, or_ref, oi_ref):
    zr = zr_ref[0]
    zi = zi_ref[0]
    C, D = zr.shape
    wqr = wqr_ref[...]
    wqi = wqi_ref[...]
    qr = _dot_t1(zr, wqr) - _dot_t1(zi, wqi)
    qi = _dot_t1(zr, wqi) + _dot_t1(zi, wqr)

    kr = kr_ref[0]
    ki = ki_ref[0]
    dot = _dot_t1(qr, kr) + _dot_t1(qi, ki)
    qmag = jnp.sqrt(jnp.sum(qr * qr + qi * qi, axis=1, keepdims=True) + 1e-8)
    kk = kr * kr + ki * ki
    kmag = jnp.sqrt(_dot_t1_hi(jnp.ones((1, D), _F32), kk) + 1e-8)
    scores = dot / (qmag * kmag + 1e-8)
    scores = jnp.where(mask_ref[0] == 0.0, -1e9, scores)

    iota_s = jax.lax.broadcasted_iota(jnp.int32, (C, S), 1)
    work = scores
    sel = []
    onehots = []
    for _ in range(TOPK):
        m = jnp.max(work, axis=1, keepdims=True)
        ismax = work == m
        idx = jnp.min(jnp.where(ismax, iota_s, S), axis=1, keepdims=True)
        oh = iota_s == idx
        sel.append(m)
        onehots.append(oh.astype(_F32))
        work = jnp.where(oh, -3.4e38, work)

    sel8 = jnp.concatenate(sel, axis=1)
    mx = jnp.max(sel8, axis=1, keepdims=True)
    e = jnp.exp(sel8 - mx)
    wts = e / jnp.sum(e, axis=1, keepdims=True)
    attn = wts[:, 0:1] * onehots[0]
    for j in range(1, TOPK):
        attn = attn + wts[:, j:j + 1] * onehots[j]

    retr = _dot_hi(attn, vr_ref[0])
    reti = _dot_hi(attn, vi_ref[0])
    rms = jnp.sqrt(jnp.mean(retr * retr + reti * reti, axis=1, keepdims=True)
                   + 1e-8)
    g = g_ref[...]
    or_ref[0] = retr / rms * g
    oi_ref[0] = reti / rms * g


@jax.jit
def kernel(z, Ws_r, Ws_i, Wk_r, Wk_i, Wv_r, Wv_i, Wq_r, Wq_i,
           score_bias, novelty_scale, gamma):
    B, L, D, _ = z.shape
    zr = z[..., 0]
    zi = z[..., 1]
    ws8 = jnp.concatenate([Ws_r, Ws_i, jnp.zeros((6, D), _F32)], axis=0)
    sb = jnp.reshape(score_bias, (1, 1)).astype(_F32)
    ns = jnp.reshape(novelty_scale, (1, 1)).astype(_F32)
    g2 = jnp.reshape(gamma, (1, D)).astype(_F32)

    full = lambda shape: pl.BlockSpec(shape, lambda b: (0,) * len(shape))
    row3 = lambda shape: pl.BlockSpec(shape, lambda b: (b, 0, 0))

    sal3, nkr, nki, nvr, nvi, mask3 = pl.pallas_call(
        _events_body,
        grid=(B,),
        in_specs=[
            row3((1, L, D)), row3((1, L, D)),
            full((8, D)),
            full((D, D)), full((D, D)), full((D, D)), full((D, D)),
            full((1, 1)), full((1, 1)),
        ],
        out_specs=[
            row3((1, L, 1)),
            row3((1, S, D)), row3((1, S, D)),
            row3((1, S, D)), row3((1, S, D)),
            row3((1, 1, S)),
        ],
        out_shape=[
            jax.ShapeDtypeStruct((B, L, 1), _F32),
            jax.ShapeDtypeStruct((B, S, D), _F32),
            jax.ShapeDtypeStruct((B, S, D), _F32),
            jax.ShapeDtypeStruct((B, S, D), _F32),
            jax.ShapeDtypeStruct((B, S, D), _F32),
            jax.ShapeDtypeStruct((B, 1, S), _F32),
        ],
    )(zr, zi, ws8, Wk_r, Wk_i, Wv_r, Wv_i, sb, ns)

    nc = L // CHUNK
    chunk3 = lambda shape: pl.BlockSpec(shape, lambda b, c: (b, c, 0))
    bcast3 = lambda shape: pl.BlockSpec(shape, lambda b, c: (b, 0, 0))
    full2 = lambda shape: pl.BlockSpec(shape, lambda b, c: (0,) * len(shape))

    out_r, out_i = pl.pallas_call(
        _read_body,
        grid=(B, nc),
        in_specs=[
            chunk3((1, CHUNK, D)), chunk3((1, CHUNK, D)),
            full2((D, D)), full2((D, D)),
            bcast3((1, S, D)), bcast3((1, S, D)),
            bcast3((1, S, D)), bcast3((1, S, D)),
            bcast3((1, 1, S)),
            full2((1, D)),
        ],
        out_specs=[
            chunk3((1, CHUNK, D)), chunk3((1, CHUNK, D)),
        ],
        out_shape=[
            jax.ShapeDtypeStruct((B, L, D), _F32),
            jax.ShapeDtypeStruct((B, L, D), _F32),
        ],
    )(zr, zi, Wq_r, Wq_i, nkr, nki, nvr, nvi, mask3, g2)

    out = jnp.stack([out_r, out_i], axis=-1)
    new_keys = jnp.stack([nkr, nki], axis=-1)
    new_values = jnp.stack([nvr, nvi], axis=-1)
    new_mask = mask3[:, 0, :]
    salience = sal3[:, :, 0]
    return out, new_keys, new_values, new_mask, salience
